# baseline (speedup 1.0000x reference)
import jax
import jax.numpy as jnp
from jax.experimental import pallas as pl

N_EMBEDDINGS = 8192
EMBEDDING_DIM = 64
BETA = 0.25

M_BLK = 1024
C_BLK = 2048
N_CHUNKS = N_EMBEDDINGS // C_BLK


def _rne_bf16(x):
    u = jax.lax.bitcast_convert_type(x, jnp.uint32)
    r = (u + jnp.uint32(0x7FFF) + ((u >> 16) & jnp.uint32(1))) \
        & jnp.uint32(0xFFFF0000)
    return jax.lax.bitcast_convert_type(r, jnp.float32)


def _vq_kernel(z_ref, cb_ref, s1_ref, s2_ref, out_ref, loss_ref):
    i = pl.program_id(0)
    z = z_ref[...]
    s1 = s1_ref[...]

    run_min = jnp.full((M_BLK,), jnp.inf, dtype=jnp.float32)
    z_q = jnp.zeros((M_BLK, EMBEDDING_DIM), dtype=jnp.float32)
    col_iota = jax.lax.broadcasted_iota(jnp.int32, (M_BLK, C_BLK), 1)

    args = []
    for k in range(N_CHUNKS):
        rows = pl.ds(k * C_BLK, C_BLK)
        cb = cb_ref[rows, :]
        s2 = s2_ref[0, rows]
        mm = jnp.dot(z, cb.T, preferred_element_type=jnp.float32)
        d = (s1 + s2[None, :]) - 2.0 * mm
        m_k = jnp.min(d, axis=1)
        a_k = jnp.min(
            jnp.where(d == m_k[:, None], col_iota, N_EMBEDDINGS), axis=1)
        better = m_k < run_min
        run_min = jnp.where(better, _rne_bf16(m_k), run_min)
        args.append((better, a_k))

    win_chunk = jnp.zeros((M_BLK,), dtype=jnp.int32)
    win_arg = jnp.zeros((M_BLK,), dtype=jnp.int32)
    for k, (better, a_k) in enumerate(args):
        win_chunk = jnp.where(better, k, win_chunk)
        win_arg = jnp.where(better, a_k, win_arg)

    for k in range(N_CHUNKS):
        rows = pl.ds(k * C_BLK, C_BLK)
        sel = (win_chunk == k)
        onehot = (jnp.where(sel, win_arg, -1)[:, None] == col_iota)
        cand = jnp.dot(onehot.astype(jnp.float32), cb_ref[rows, :],
                       preferred_element_type=jnp.float32,
                       precision=jax.lax.Precision.HIGHEST)
        z_q = jnp.where(sel[:, None], cand, z_q)

    out_ref[...] = z + (z_q - z)

    @pl.when(i == 0)
    def _init():
        loss_ref[...] = jnp.zeros((1, 1), jnp.float32)

    loss_ref[...] += jnp.sum((z_q - z) ** 2).reshape(1, 1)


@jax.jit
def kernel(z, codebook):
    z_flat = z.reshape(-1, EMBEDDING_DIM)
    n_rows = z_flat.shape[0]
    s1 = jnp.sum(z_flat ** 2, axis=1, keepdims=True)
    s2 = jnp.sum(codebook ** 2, axis=1).reshape(1, -1)
    grid = (n_rows // M_BLK,)
    out, loss_sum = pl.pallas_call(
        _vq_kernel,
        grid=grid,
        in_specs=[
            pl.BlockSpec((M_BLK, EMBEDDING_DIM), lambda i: (i, 0)),
            pl.BlockSpec((N_EMBEDDINGS, EMBEDDING_DIM), lambda i: (0, 0)),
            pl.BlockSpec((M_BLK, 1), lambda i: (i, 0)),
            pl.BlockSpec((1, N_EMBEDDINGS), lambda i: (0, 0)),
        ],
        out_specs=[
            pl.BlockSpec((M_BLK, EMBEDDING_DIM), lambda i: (i, 0)),
            pl.BlockSpec((1, 1), lambda i: (0, 0)),
        ],
        out_shape=[
            jax.ShapeDtypeStruct((n_rows, EMBEDDING_DIM), jnp.float32),
            jax.ShapeDtypeStruct((1, 1), jnp.float32),
        ],
    )(z_flat, codebook, s1, s2)
    mean_sq = loss_sum[0, 0] / (n_rows * EMBEDDING_DIM)
    embedding_loss = mean_sq + BETA * mean_sq
    return out.reshape(z.shape), embedding_loss


# speedup vs baseline: 2.0760x; 2.0760x over previous
import jax
import jax.numpy as jnp
from jax.experimental import pallas as pl

N_EMBEDDINGS = 8192
EMBEDDING_DIM = 64
BETA = 0.25

M_BLK = 1024
C_BLK = 2048
N_CHUNKS = N_EMBEDDINGS // C_BLK
KPACK = 4
GDIV = 4


def _rne_bf16(x):
    u = jax.lax.bitcast_convert_type(x, jnp.uint32)
    r = (u + jnp.uint32(0x7FFF) + ((u >> 16) & jnp.uint32(1))) \
        & jnp.uint32(0xFFFF0000)
    return jax.lax.bitcast_convert_type(r, jnp.float32)


def _vq_kernel(z_ref, bm2_ref, g_hi_ref, g_mid_ref, g_lo_ref, s1_ref, s2_ref,
               out_ref, loss_ref):
    i = pl.program_id(0)
    z = z_ref[...]
    s1 = s1_ref[...]
    z4 = jnp.concatenate([z] * KPACK, axis=1)

    run_min = jnp.full((M_BLK,), jnp.inf, dtype=jnp.float32)
    col_iota = jax.lax.broadcasted_iota(jnp.int32, (M_BLK, C_BLK), 1)

    args = []
    for k in range(N_CHUNKS):
        bm2 = bm2_ref[:, pl.ds(k * C_BLK, C_BLK)]
        s2 = s2_ref[0, pl.ds(k * C_BLK, C_BLK)]
        mm2 = jnp.dot(z4, bm2, preferred_element_type=jnp.float32)
        d = (s1 + s2[None, :]) + mm2
        m_k = jnp.min(d, axis=1)
        a_k = jnp.min(
            jnp.where(d == m_k[:, None], col_iota, N_EMBEDDINGS), axis=1)
        better = m_k < run_min
        run_min = jnp.where(better, _rne_bf16(m_k), run_min)
        args.append((better, a_k))

    win_chunk = jnp.zeros((M_BLK,), dtype=jnp.int32)
    win_arg = jnp.zeros((M_BLK,), dtype=jnp.int32)
    for k, (better, a_k) in enumerate(args):
        win_chunk = jnp.where(better, k, win_chunk)
        win_arg = jnp.where(better, a_k + k * C_BLK, win_arg)

    grp = win_arg // GDIV
    onehot = (grp[:, None] == col_iota).astype(jnp.bfloat16)
    out4 = (jnp.dot(onehot, g_hi_ref[...], preferred_element_type=jnp.float32)
            + jnp.dot(onehot, g_mid_ref[...], preferred_element_type=jnp.float32)
            + jnp.dot(onehot, g_lo_ref[...], preferred_element_type=jnp.float32))
    rem = win_arg % GDIV
    z_q = out4[:, 0:EMBEDDING_DIM]
    for p in range(1, GDIV):
        z_q = jnp.where((rem == p)[:, None],
                        out4[:, p * EMBEDDING_DIM:(p + 1) * EMBEDDING_DIM],
                        z_q)

    out_ref[...] = z + (z_q - z)

    @pl.when(i == 0)
    def _init():
        loss_ref[...] = jnp.zeros((1, 1), jnp.float32)

    loss_ref[...] += jnp.sum((z_q - z) ** 2).reshape(1, 1)


@jax.jit
def kernel(z, codebook):
    z_flat = z.reshape(-1, EMBEDDING_DIM)
    n_rows = z_flat.shape[0]
    s1 = jnp.sum(z_flat ** 2, axis=1, keepdims=True)
    s2 = jnp.sum(codebook ** 2, axis=1).reshape(1, -1)
    cbm2_t = (-2.0 * codebook).T
    sel = (jnp.arange(N_EMBEDDINGS) % KPACK)[None, :] \
        == jnp.arange(KPACK)[:, None]
    bm2 = (sel[:, None, :] * cbm2_t[None]).reshape(
        KPACK * EMBEDDING_DIM, N_EMBEDDINGS)
    cb_hi = codebook.astype(jnp.bfloat16)
    r1 = codebook - cb_hi.astype(jnp.float32)
    cb_mid = r1.astype(jnp.bfloat16)
    cb_lo = (r1 - cb_mid.astype(jnp.float32)).astype(jnp.bfloat16)
    gshape = (N_EMBEDDINGS // GDIV, GDIV * EMBEDDING_DIM)
    g_hi = cb_hi.reshape(gshape)
    g_mid = cb_mid.reshape(gshape)
    g_lo = cb_lo.reshape(gshape)

    grid = (n_rows // M_BLK,)
    out, loss_sum = pl.pallas_call(
        _vq_kernel,
        grid=grid,
        in_specs=[
            pl.BlockSpec((M_BLK, EMBEDDING_DIM), lambda i: (i, 0)),
            pl.BlockSpec(bm2.shape, lambda i: (0, 0)),
            pl.BlockSpec(gshape, lambda i: (0, 0)),
            pl.BlockSpec(gshape, lambda i: (0, 0)),
            pl.BlockSpec(gshape, lambda i: (0, 0)),
            pl.BlockSpec((M_BLK, 1), lambda i: (i, 0)),
            pl.BlockSpec((1, N_EMBEDDINGS), lambda i: (0, 0)),
        ],
        out_specs=[
            pl.BlockSpec((M_BLK, EMBEDDING_DIM), lambda i: (i, 0)),
            pl.BlockSpec((1, 1), lambda i: (0, 0)),
        ],
        out_shape=[
            jax.ShapeDtypeStruct((n_rows, EMBEDDING_DIM), jnp.float32),
            jax.ShapeDtypeStruct((1, 1), jnp.float32),
        ],
    )(z_flat, bm2, g_hi, g_mid, g_lo, s1, s2)
    mean_sq = loss_sum[0, 0] / (n_rows * EMBEDDING_DIM)
    embedding_loss = mean_sq + BETA * mean_sq
    return out.reshape(z.shape), embedding_loss
